# Initial kernel scaffold; baseline (speedup 1.0000x reference)
import jax
import jax.numpy as jnp
from jax.experimental import pallas as pl


def kernel(x, x_len):
    raise NotImplementedError("write your pallas kernel here")


# speedup vs baseline: 1.0066x; 1.0066x over previous
import jax
import jax.numpy as jnp
from jax.experimental import pallas as pl
from jax.experimental.pallas import tpu as pltpu

_ROWS = 16 * 2048
_COLS = 1024
_BLOCK_ROWS = 2048


def _copy_body(x_ref, len_ref, x_out_ref, len_out_ref):
    x_out_ref[...] = x_ref[...]
    len_out_ref[...] = len_ref[...]


def kernel(x, x_len):
    x2 = x.reshape(_ROWS, _COLS)
    len2 = x_len.reshape(1, 16)
    out_x, out_len = pl.pallas_call(
        _copy_body,
        grid=(_ROWS // _BLOCK_ROWS,),
        in_specs=[
            pl.BlockSpec((_BLOCK_ROWS, _COLS), lambda i: (i, 0)),
            pl.BlockSpec((1, 16), lambda i: (0, 0)),
        ],
        out_specs=[
            pl.BlockSpec((_BLOCK_ROWS, _COLS), lambda i: (i, 0)),
            pl.BlockSpec((1, 16), lambda i: (0, 0)),
        ],
        out_shape=[
            jax.ShapeDtypeStruct((_ROWS, _COLS), x.dtype),
            jax.ShapeDtypeStruct((1, 16), x_len.dtype),
        ],
        compiler_params=pltpu.CompilerParams(
            dimension_semantics=("arbitrary",),
        ),
    )(x2, len2)
    return out_x.reshape(x.shape), out_len.reshape(x_len.shape)


# speedup vs baseline: 1.0070x; 1.0004x over previous
import jax
import jax.numpy as jnp
from jax.experimental import pallas as pl
from jax.experimental.pallas import tpu as pltpu

_ROWS = 16 * 2048
_COLS = 1024
_BLOCK_ROWS = 2048


def _copy_body(x_ref, len_ref, x_out_ref, len_out_ref):
    x_out_ref[...] = x_ref[...]
    len_out_ref[...] = len_ref[...]


def kernel(x, x_len):
    x2 = x.reshape(_ROWS, _COLS)
    len2 = x_len.reshape(1, 16)
    out_x, out_len = pl.pallas_call(
        _copy_body,
        grid=(_ROWS // _BLOCK_ROWS,),
        in_specs=[
            pl.BlockSpec((_BLOCK_ROWS, _COLS), lambda i: (i, 0)),
            pl.BlockSpec((1, 16), lambda i: (0, 0)),
        ],
        out_specs=[
            pl.BlockSpec((_BLOCK_ROWS, _COLS), lambda i: (i, 0)),
            pl.BlockSpec((1, 16), lambda i: (0, 0)),
        ],
        out_shape=[
            jax.ShapeDtypeStruct((_ROWS, _COLS), x.dtype),
            jax.ShapeDtypeStruct((1, 16), x_len.dtype),
        ],
        compiler_params=pltpu.CompilerParams(
            dimension_semantics=("parallel",),
        ),
    )(x2, len2)
    return out_x.reshape(x.shape), out_len.reshape(x_len.shape)
